# baseline (speedup 1.0000x reference)
import jax
import jax.numpy as jnp
from jax import lax
from jax.experimental import pallas as pl
from jax.experimental.pallas import tpu as pltpu
from jax.experimental.pallas import tpu_sc as plsc

B, L, D, C = 1024, 200, 50000, 2

_info = plsc.get_sparse_core_info()
NC, NS, LANES = _info.num_cores, _info.num_subcores, _info.num_lanes
NW = NC * NS
ROWS_PER_W = B // NW
GROUPS = ROWS_PER_W // LANES
TOK_PER_W = ROWS_PER_W * L
DP = D + 8


def _sc_kernel_body(iv_hbm, wp_hbm, out_hbm,
                    w_v, idx_v, val_v, out_v, w_sem):
    wid = lax.axis_index("s") * NC + lax.axis_index("c")
    base = wid * ROWS_PER_W

    w_cp = pltpu.async_copy(wp_hbm, w_v, w_sem)
    pltpu.sync_copy(iv_hbm.at[pl.ds(base * L, TOK_PER_W)], idx_v)
    pltpu.sync_copy(iv_hbm.at[pl.ds(B * L + base * L, TOK_PER_W)], val_v)
    w_cp.wait()

    iota = lax.iota(jnp.int32, LANES)
    hi_mask = jnp.full((LANES,), -65536, jnp.int32)
    b0 = lax.bitcast_convert_type(
        plsc.load_gather(w_v, [jnp.full((LANES,), D, jnp.int32)]), jnp.float32)
    b1 = lax.bitcast_convert_type(
        plsc.load_gather(w_v, [jnp.full((LANES,), D + 1, jnp.int32)]),
        jnp.float32)
    row_base = [(iota + g * LANES) * L for g in range(GROUPS)]
    out_pos = [(iota + g * LANES) * C for g in range(GROUPS)]

    def body(j, carry):
        out = []
        for g in range(GROUPS):
            a0, a1 = carry[2 * g], carry[2 * g + 1]
            pos = row_base[g] + j
            iv = plsc.load_gather(idx_v, [pos])
            vv = lax.bitcast_convert_type(
                plsc.load_gather(val_v, [pos]), jnp.float32)
            wp = plsc.load_gather(w_v, [iv])
            w0 = lax.bitcast_convert_type(lax.shift_left(wp, 16), jnp.float32)
            w1 = lax.bitcast_convert_type(jnp.bitwise_and(wp, hi_mask),
                                          jnp.float32)
            out.extend([a0 + vv * w0, a1 + vv * w1])
        return tuple(out)

    accs = lax.fori_loop(0, L, body, (b0, b1) * GROUPS)
    for g in range(GROUPS):
        plsc.store_scatter(out_v, [out_pos[g]], accs[2 * g])
        plsc.store_scatter(out_v, [out_pos[g] + 1], accs[2 * g + 1])

    pltpu.sync_copy(out_v, out_hbm.at[pl.ds(base * C, ROWS_PER_W * C)])


@jax.jit
def kernel(idxs, vals, W, b):
    ivpack = jnp.concatenate(
        [idxs.reshape(B * L),
         jax.lax.bitcast_convert_type(vals.reshape(B * L), jnp.int32)])
    wb = jax.lax.bitcast_convert_type(W.astype(jnp.bfloat16), jnp.uint16)
    wp = (wb[:, 0].astype(jnp.uint32)
          | (wb[:, 1].astype(jnp.uint32) << 16))
    wp = jax.lax.bitcast_convert_type(wp, jnp.int32)
    btail = jnp.pad(jax.lax.bitcast_convert_type(b, jnp.int32), (0, 6))
    wpb = jnp.concatenate([wp, btail])
    mesh = plsc.VectorSubcoreMesh(core_axis_name="c", subcore_axis_name="s")
    run = pl.kernel(
        _sc_kernel_body,
        out_type=jax.ShapeDtypeStruct((B * C,), jnp.float32),
        mesh=mesh,
        scratch_types=[
            pltpu.VMEM((DP,), jnp.int32),
            pltpu.VMEM((TOK_PER_W,), jnp.int32),
            pltpu.VMEM((TOK_PER_W,), jnp.int32),
            pltpu.VMEM((ROWS_PER_W * C,), jnp.float32),
            pltpu.SemaphoreType.DMA,
        ],
        compiler_params=pltpu.CompilerParams(
            use_tc_tiling_on_sc=False, needs_layout_passes=False
        ),
    )
    return run(ivpack, wpb).reshape(B, C)


# speedup vs baseline: 1.0171x; 1.0171x over previous
import jax
import jax.numpy as jnp
from jax import lax
from jax.experimental import pallas as pl
from jax.experimental.pallas import tpu as pltpu
from jax.experimental.pallas import tpu_sc as plsc

B, L, D, C = 1024, 200, 50000, 2

_info = plsc.get_sparse_core_info()
NC, NS, LANES = _info.num_cores, _info.num_subcores, _info.num_lanes
NW = NC * NS
ROWS_PER_W = B // NW
GROUPS = ROWS_PER_W // LANES
TOK_PER_W = ROWS_PER_W * L
DP = D + 8


def _sc_kernel_body(iv_hbm, wp_hbm, out_hbm,
                    w_v, idx_v, val_v, out_v, w_sem):
    wid = lax.axis_index("s") * NC + lax.axis_index("c")
    base = wid * ROWS_PER_W

    w_cps = []
    chunk = (DP // 4 + 7) // 8 * 8
    for k in range(4):
        lo = k * chunk
        sz = min(chunk, DP - lo)
        w_cps.append(pltpu.async_copy(
            wp_hbm.at[pl.ds(lo, sz)],
            w_v.at[pl.ds(lo, sz)], w_sem))
    pltpu.sync_copy(iv_hbm.at[pl.ds(base * L, TOK_PER_W)], idx_v)
    pltpu.sync_copy(iv_hbm.at[pl.ds(B * L + base * L, TOK_PER_W)], val_v)
    for cp in w_cps:
        cp.wait()

    iota = lax.iota(jnp.int32, LANES)
    hi_mask = jnp.full((LANES,), -65536, jnp.int32)
    b0 = lax.bitcast_convert_type(
        plsc.load_gather(w_v, [jnp.full((LANES,), D, jnp.int32)]), jnp.float32)
    b1 = lax.bitcast_convert_type(
        plsc.load_gather(w_v, [jnp.full((LANES,), D + 1, jnp.int32)]),
        jnp.float32)
    row_base = [(iota + g * LANES) * L for g in range(GROUPS)]
    out_pos = [iota + g * LANES for g in range(GROUPS)]

    def body(j, carry):
        out = []
        for g in range(GROUPS):
            a0, a1 = carry[2 * g], carry[2 * g + 1]
            pos = row_base[g] + j
            iv = plsc.load_gather(idx_v, [pos])
            vv = lax.bitcast_convert_type(
                plsc.load_gather(val_v, [pos]), jnp.float32)
            wp = plsc.load_gather(w_v, [iv])
            w0 = lax.bitcast_convert_type(lax.shift_left(wp, 16), jnp.float32)
            w1 = lax.bitcast_convert_type(jnp.bitwise_and(wp, hi_mask),
                                          jnp.float32)
            out.extend([a0 + vv * w0, a1 + vv * w1])
        return tuple(out)

    accs = lax.fori_loop(0, L, body, (b0, b1) * GROUPS)
    for g in range(GROUPS):
        plsc.store_scatter(out_v, [out_pos[g]], accs[2 * g])
        plsc.store_scatter(out_v, [out_pos[g] + ROWS_PER_W], accs[2 * g + 1])

    pltpu.sync_copy(out_v.at[pl.ds(0, ROWS_PER_W)],
                    out_hbm.at[pl.ds(base, ROWS_PER_W)])
    pltpu.sync_copy(out_v.at[pl.ds(ROWS_PER_W, ROWS_PER_W)],
                    out_hbm.at[pl.ds(B + base, ROWS_PER_W)])


@jax.jit
def kernel(idxs, vals, W, b):
    ivpack = jnp.concatenate(
        [idxs.reshape(B * L),
         jax.lax.bitcast_convert_type(vals.reshape(B * L), jnp.int32)])
    wb = jax.lax.bitcast_convert_type(W.astype(jnp.bfloat16), jnp.uint16)
    wp = (wb[:, 0].astype(jnp.uint32)
          | (wb[:, 1].astype(jnp.uint32) << 16))
    wp = jax.lax.bitcast_convert_type(wp, jnp.int32)
    btail = jnp.pad(jax.lax.bitcast_convert_type(b, jnp.int32), (0, 6))
    wpb = jnp.concatenate([wp, btail])
    mesh = plsc.VectorSubcoreMesh(core_axis_name="c", subcore_axis_name="s")
    run = pl.kernel(
        _sc_kernel_body,
        out_type=jax.ShapeDtypeStruct((C * B,), jnp.float32),
        mesh=mesh,
        scratch_types=[
            pltpu.VMEM((DP,), jnp.int32),
            pltpu.VMEM((TOK_PER_W,), jnp.int32),
            pltpu.VMEM((TOK_PER_W,), jnp.int32),
            pltpu.VMEM((ROWS_PER_W * C,), jnp.float32),
            pltpu.SemaphoreType.DMA,
        ],
        compiler_params=pltpu.CompilerParams(
            use_tc_tiling_on_sc=False, needs_layout_passes=False
        ),
    )
    return run(ivpack, wpb).reshape(C, B).T


# speedup vs baseline: 1.0217x; 1.0045x over previous
import jax
import jax.numpy as jnp
from jax import lax
from jax.experimental import pallas as pl
from jax.experimental.pallas import tpu as pltpu
from jax.experimental.pallas import tpu_sc as plsc

B, L, D, C = 1024, 200, 50000, 2

_info = plsc.get_sparse_core_info()
NC, NS, LANES = _info.num_cores, _info.num_subcores, _info.num_lanes
NW = NC * NS
ROWS_PER_W = B // NW
GROUPS = ROWS_PER_W // LANES
TOK_PER_W = ROWS_PER_W * L
DP = D + 8


def _sc_kernel_body(iv_hbm, wp_hbm, out_hbm,
                    w_v, idx_v, val_v, out_v, w_sem):
    wid = lax.axis_index("s") * NC + lax.axis_index("c")
    base = wid * ROWS_PER_W

    w_cps = []
    chunk = (DP // 4 + 7) // 8 * 8
    for k in range(4):
        lo = k * chunk
        sz = min(chunk, DP - lo)
        w_cps.append(pltpu.async_copy(
            wp_hbm.at[pl.ds(lo, sz)],
            w_v.at[pl.ds(lo, sz)], w_sem))
    pltpu.sync_copy(iv_hbm.at[pl.ds(base * L, TOK_PER_W)], idx_v)
    pltpu.sync_copy(iv_hbm.at[pl.ds(B * L + base * L, TOK_PER_W)], val_v)
    for cp in w_cps:
        cp.wait()

    iota = lax.iota(jnp.int32, LANES)
    hi_mask = jnp.full((LANES,), -65536, jnp.int32)
    b0 = lax.bitcast_convert_type(
        plsc.load_gather(w_v, [jnp.full((LANES,), D, jnp.int32)]), jnp.float32)
    b1 = lax.bitcast_convert_type(
        plsc.load_gather(w_v, [jnp.full((LANES,), D + 1, jnp.int32)]),
        jnp.float32)
    row_base = [(iota + g * LANES) * L for g in range(GROUPS)]
    out_pos = [iota + g * LANES for g in range(GROUPS)]

    UNROLL = 2
    zero = jnp.zeros((LANES,), jnp.float32)

    def body(j, carry):
        out = []
        k = 0
        for g in range(GROUPS):
            for u in range(UNROLL):
                a0, a1 = carry[k], carry[k + 1]
                pos = row_base[g] + (j + u)
                iv = plsc.load_gather(idx_v, [pos])
                vv = lax.bitcast_convert_type(
                    plsc.load_gather(val_v, [pos]), jnp.float32)
                wp = plsc.load_gather(w_v, [iv])
                w0 = lax.bitcast_convert_type(lax.shift_left(wp, 16),
                                              jnp.float32)
                w1 = lax.bitcast_convert_type(jnp.bitwise_and(wp, hi_mask),
                                              jnp.float32)
                out.extend([a0 + vv * w0, a1 + vv * w1])
                k += 2
        return tuple(out)

    init = []
    for g in range(GROUPS):
        init.extend([b0, b1] + [zero, zero] * (UNROLL - 1))
    accs = lax.fori_loop(0, L // UNROLL, lambda i, c: body(i * UNROLL, c),
                         tuple(init))
    for g in range(GROUPS):
        k = 2 * UNROLL * g
        a0 = accs[k]
        a1 = accs[k + 1]
        for u in range(1, UNROLL):
            a0 = a0 + accs[k + 2 * u]
            a1 = a1 + accs[k + 2 * u + 1]
        plsc.store_scatter(out_v, [out_pos[g]], a0)
        plsc.store_scatter(out_v, [out_pos[g] + ROWS_PER_W], a1)

    pltpu.sync_copy(out_v.at[pl.ds(0, ROWS_PER_W)],
                    out_hbm.at[pl.ds(base, ROWS_PER_W)])
    pltpu.sync_copy(out_v.at[pl.ds(ROWS_PER_W, ROWS_PER_W)],
                    out_hbm.at[pl.ds(B + base, ROWS_PER_W)])


@jax.jit
def kernel(idxs, vals, W, b):
    ivpack = jnp.concatenate(
        [idxs.reshape(B * L),
         jax.lax.bitcast_convert_type(vals.reshape(B * L), jnp.int32)])
    wb = jax.lax.bitcast_convert_type(W.astype(jnp.bfloat16), jnp.uint16)
    wp = (wb[:, 0].astype(jnp.uint32)
          | (wb[:, 1].astype(jnp.uint32) << 16))
    wp = jax.lax.bitcast_convert_type(wp, jnp.int32)
    btail = jnp.pad(jax.lax.bitcast_convert_type(b, jnp.int32), (0, 6))
    wpb = jnp.concatenate([wp, btail])
    mesh = plsc.VectorSubcoreMesh(core_axis_name="c", subcore_axis_name="s")
    run = pl.kernel(
        _sc_kernel_body,
        out_type=jax.ShapeDtypeStruct((C * B,), jnp.float32),
        mesh=mesh,
        scratch_types=[
            pltpu.VMEM((DP,), jnp.int32),
            pltpu.VMEM((TOK_PER_W,), jnp.int32),
            pltpu.VMEM((TOK_PER_W,), jnp.int32),
            pltpu.VMEM((ROWS_PER_W * C,), jnp.float32),
            pltpu.SemaphoreType.DMA,
        ],
        compiler_params=pltpu.CompilerParams(
            use_tc_tiling_on_sc=False, needs_layout_passes=False
        ),
    )
    return run(ivpack, wpb).reshape(C, B).T


# speedup vs baseline: 1.1228x; 1.0989x over previous
import jax
import jax.numpy as jnp
from jax import lax
from jax.experimental import pallas as pl
from jax.experimental.pallas import tpu as pltpu
from jax.experimental.pallas import tpu_sc as plsc

B, L, D, C = 1024, 200, 50000, 2

_info = plsc.get_sparse_core_info()
NC, NS, LANES = _info.num_cores, _info.num_subcores, _info.num_lanes
NW = NC * NS
ROWS_PER_W = B // NW
GROUPS = ROWS_PER_W // LANES
TOK_PER_W = ROWS_PER_W * L
DP = D + 8
GCHUNK = 128
NCHUNK = TOK_PER_W // GCHUNK


def _sc_kernel_body(iv_hbm, wp_hbm, out_hbm,
                    w_sh, idx_v, val_v, wtok_v, out_v, b0_v, b1_v,
                    w_sem, g_sem):
    sid = lax.axis_index("s")
    wid = sid * NC + lax.axis_index("c")
    base = wid * ROWS_PER_W

    @pl.when(sid == 0)
    def _():
        pltpu.async_copy(wp_hbm, w_sh, w_sem).wait()

    pltpu.sync_copy(iv_hbm.at[pl.ds(base * L, TOK_PER_W)], idx_v)
    pltpu.sync_copy(iv_hbm.at[pl.ds(B * L + base * L, TOK_PER_W)], val_v)
    plsc.subcore_barrier()

    cps = []
    for k in range(NCHUNK):
        cps.append(pltpu.async_copy(
            w_sh.at[idx_v.at[pl.ds(k * GCHUNK, GCHUNK)]],
            wtok_v.at[pl.ds(k * GCHUNK, GCHUNK)], g_sem))
    cps.append(pltpu.async_copy(
        w_sh.at[jnp.full((LANES,), D, jnp.int32)], b0_v, g_sem))
    cps.append(pltpu.async_copy(
        w_sh.at[jnp.full((LANES,), D + 1, jnp.int32)], b1_v, g_sem))
    for cp in cps:
        cp.wait()

    iota = lax.iota(jnp.int32, LANES)
    hi_mask = jnp.full((LANES,), -65536, jnp.int32)
    b0 = lax.bitcast_convert_type(b0_v[...], jnp.float32)
    b1 = lax.bitcast_convert_type(b1_v[...], jnp.float32)
    row_base = [(iota + g * LANES) * L for g in range(GROUPS)]
    out_pos = [iota + g * LANES for g in range(GROUPS)]

    def body(j, carry):
        out = []
        for g in range(GROUPS):
            a0, a1 = carry[2 * g], carry[2 * g + 1]
            pos = row_base[g] + j
            vv = lax.bitcast_convert_type(
                plsc.load_gather(val_v, [pos]), jnp.float32)
            wp = plsc.load_gather(wtok_v, [pos])
            w0 = lax.bitcast_convert_type(lax.shift_left(wp, 16), jnp.float32)
            w1 = lax.bitcast_convert_type(jnp.bitwise_and(wp, hi_mask),
                                          jnp.float32)
            out.extend([a0 + vv * w0, a1 + vv * w1])
        return tuple(out)

    accs = lax.fori_loop(0, L, body, (b0, b1) * GROUPS)
    for g in range(GROUPS):
        plsc.store_scatter(out_v, [out_pos[g]], accs[2 * g])
        plsc.store_scatter(out_v, [out_pos[g] + ROWS_PER_W], accs[2 * g + 1])

    pltpu.sync_copy(out_v.at[pl.ds(0, ROWS_PER_W)],
                    out_hbm.at[pl.ds(base, ROWS_PER_W)])
    pltpu.sync_copy(out_v.at[pl.ds(ROWS_PER_W, ROWS_PER_W)],
                    out_hbm.at[pl.ds(B + base, ROWS_PER_W)])


@jax.jit
def kernel(idxs, vals, W, b):
    ivpack = jnp.concatenate(
        [idxs.reshape(B * L),
         jax.lax.bitcast_convert_type(vals.reshape(B * L), jnp.int32)])
    wb = jax.lax.bitcast_convert_type(W.astype(jnp.bfloat16), jnp.uint16)
    wp = (wb[:, 0].astype(jnp.uint32)
          | (wb[:, 1].astype(jnp.uint32) << 16))
    wp = jax.lax.bitcast_convert_type(wp, jnp.int32)
    btail = jnp.pad(jax.lax.bitcast_convert_type(b, jnp.int32), (0, 6))
    wpb = jnp.concatenate([wp, btail])
    mesh = plsc.VectorSubcoreMesh(core_axis_name="c", subcore_axis_name="s")
    run = pl.kernel(
        _sc_kernel_body,
        out_type=jax.ShapeDtypeStruct((C * B,), jnp.float32),
        mesh=mesh,
        scratch_types=[
            pltpu.VMEM_SHARED((DP,), jnp.int32),
            pltpu.VMEM((TOK_PER_W,), jnp.int32),
            pltpu.VMEM((TOK_PER_W,), jnp.int32),
            pltpu.VMEM((TOK_PER_W,), jnp.int32),
            pltpu.VMEM((ROWS_PER_W * C,), jnp.float32),
            pltpu.VMEM((LANES,), jnp.int32),
            pltpu.VMEM((LANES,), jnp.int32),
            pltpu.SemaphoreType.DMA,
            pltpu.SemaphoreType.DMA,
        ],
        compiler_params=pltpu.CompilerParams(
            use_tc_tiling_on_sc=False, needs_layout_passes=False
        ),
    )
    return run(ivpack, wpb).reshape(C, B).T
